# baseline (speedup 1.0000x reference)
import functools

import jax
import jax.numpy as jnp
from jax import lax
from jax.experimental import pallas as pl
from jax.experimental.pallas import tpu as pltpu
from jax.experimental.pallas import tpu_sc as plsc

N = 10000
E = 320000
F = 128
HID = 32
P = 12

NC = 2
NS = 16

CW = 128
CH = (2 * HID * P) // CW
EB = 80
EPAD = 332800
TE_SC = EPAD // NS
TE_A = E // (NC * NS)
NPAD = 10240
RPT = NPAD // NS

_mesh = plsc.VectorSubcoreMesh(
    core_axis_name="c", subcore_axis_name="s", num_cores=NC, num_subcores=NS)


DW = 128
EBA = 80
NB_A = TE_A // EBA


def _deg_body(dst_hbm, ones_hbm, z_hbm, out_hbm, dstb, dstc, onesb, sdeg):
    c = lax.axis_index("c")
    s = lax.axis_index("s")
    w = s * NC + c
    pltpu.sync_copy(dst_hbm.at[pl.ds(w * TE_A, TE_A)], dstb)
    pltpu.sync_copy(ones_hbm, onesb)
    pltpu.sync_copy(z_hbm, sdeg.at[pl.ds(s * RPT, RPT)])
    plsc.subcore_barrier()

    def astep(b, carry):
        for k in range(EBA // 16):
            dstc[pl.ds(k * 16, 16)] = dstb[pl.ds(b * EBA + k * 16, 16)]
        pltpu.sync_copy(onesb, sdeg.at[dstc], add=True)
        return carry

    lax.fori_loop(0, NB_A, astep, 0)
    plsc.subcore_barrier()
    pltpu.sync_copy(sdeg.at[pl.ds(s * RPT, RPT)],
                    out_hbm.at[pl.ds(c * NPAD + s * RPT, RPT)])


_deg_call = pl.kernel(
    _deg_body,
    out_type=jax.ShapeDtypeStruct((NC * NPAD, DW), jnp.float32),
    mesh=_mesh,
    scratch_types=[
        pltpu.VMEM((TE_A,), jnp.int32),
        pltpu.VMEM((EBA,), jnp.int32),
        pltpu.VMEM((EBA, DW), jnp.float32),
        pltpu.VMEM_SHARED((NPAD, DW), jnp.float32),
    ],
)


def _dis_body(deg_ref, dis_ref):
    deg = deg_ref[0, 0:N, 0:1] + deg_ref[1, 0:N, 0:1]
    dis_ref[...] = lax.rsqrt(deg + 1.0)


_dis_call = pl.pallas_call(
    _dis_body,
    out_shape=jax.ShapeDtypeStruct((N, 1), jnp.float32),
)


_BN = 1000


def _proj_body(x_ref, dis_ref, Wz_ref, Wh_ref, LzW_ref, LhW_ref, y_ref):
    f32 = jnp.float32
    Wz2 = jnp.dot(Wz_ref[...], LzW_ref[0:HID, :], preferred_element_type=f32)
    Wh2 = jnp.dot(Wh_ref[...], LhW_ref[0:HID, :], preferred_element_type=f32)
    dis = dis_ref[...]
    zs = [jnp.dot(x_ref[p], Wz2, preferred_element_type=f32) for p in range(P)]
    hs = [jnp.dot(x_ref[p], Wh2, preferred_element_type=f32) for p in range(P)]
    for i in range(CH // 2):
        y_ref[i] = dis * jnp.concatenate(zs[4 * i:4 * i + 4], axis=1)
        y_ref[CH // 2 + i] = dis * jnp.concatenate(hs[4 * i:4 * i + 4], axis=1)


_proj_call = pl.pallas_call(
    _proj_body,
    grid=(N // _BN,),
    in_specs=[
        pl.BlockSpec((P, _BN, F), lambda i: (0, i, 0)),
        pl.BlockSpec((_BN, 1), lambda i: (i, 0)),
        pl.BlockSpec((F, HID), lambda i: (0, 0)),
        pl.BlockSpec((F, HID), lambda i: (0, 0)),
        pl.BlockSpec((2 * HID, HID), lambda i: (0, 0)),
        pl.BlockSpec((2 * HID, HID), lambda i: (0, 0)),
    ],
    out_specs=pl.BlockSpec((CH, _BN, CW), lambda i: (0, i, 0)),
    out_shape=jax.ShapeDtypeStruct((CH, NPAD, CW), jnp.float32),
)


SB = 4160
NSB = TE_SC // SB
NB_I = SB // EB
NPAIR = NB_I // 2
NACC = NPAD + 8


def _agg_body(y_hbm, src_hbm, dst_hbm, z_hbm, out_hbm,
              srcb, dstb, rf0, rf1, idx0, idx1, dstc0, dstc1,
              ssh, sem0, sem1):
    c = lax.axis_index("c")
    s = lax.axis_index("s")
    for q in range(CH // NC):
        chunk = c * (CH // NC) + q
        off = chunk * NPAD
        pltpu.sync_copy(z_hbm, ssh.at[pl.ds(s * RPT, RPT)])
        plsc.subcore_barrier()

        def fill(b, idxr, dstcr):
            for k in range(EB // 16):
                idxr[pl.ds(k * 16, 16)] = srcb[pl.ds(b * EB + k * 16, 16)] + off
                dstcr[pl.ds(k * 16, 16)] = dstb[pl.ds(b * EB + k * 16, 16)]

        def sbstep(sb, carry):
            base = s * TE_SC + sb * SB
            pltpu.sync_copy(src_hbm.at[pl.ds(base, SB)], srcb)
            pltpu.sync_copy(dst_hbm.at[pl.ds(base, SB)], dstb)
            fill(0, idx0, dstc0)
            pltpu.async_copy(y_hbm.at[idx0], rf0, sem0)

            def pair(bb, carry2):
                b0 = bb * 2
                fill(b0 + 1, idx1, dstc1)
                pltpu.async_copy(y_hbm.at[idx1], rf1, sem1)
                pltpu.make_async_copy(y_hbm.at[idx0], rf0, sem0).wait()
                pltpu.sync_copy(rf0, ssh.at[dstc0], add=True)

                @pl.when(bb < NPAIR - 1)
                def _():
                    fill(b0 + 2, idx0, dstc0)
                    pltpu.async_copy(y_hbm.at[idx0], rf0, sem0)

                pltpu.make_async_copy(y_hbm.at[idx1], rf1, sem1).wait()
                pltpu.sync_copy(rf1, ssh.at[dstc1], add=True)
                return carry2

            lax.fori_loop(0, NPAIR, pair, 0)
            return carry

        lax.fori_loop(0, NSB, sbstep, 0)
        plsc.subcore_barrier()
        pltpu.sync_copy(ssh.at[pl.ds(s * RPT, RPT)],
                        out_hbm.at[pl.ds(chunk * NPAD + s * RPT, RPT)])
        plsc.subcore_barrier()


_agg_call = pl.kernel(
    _agg_body,
    out_type=jax.ShapeDtypeStruct((CH * NPAD, CW), jnp.float32),
    mesh=_mesh,
    scratch_types=[
        pltpu.VMEM((SB,), jnp.int32),
        pltpu.VMEM((SB,), jnp.int32),
        pltpu.VMEM((EB, CW), jnp.float32),
        pltpu.VMEM((EB, CW), jnp.float32),
        pltpu.VMEM((EB,), jnp.int32),
        pltpu.VMEM((EB,), jnp.int32),
        pltpu.VMEM((EB,), jnp.int32),
        pltpu.VMEM((EB,), jnp.int32),
        pltpu.VMEM_SHARED((NACC, CW), jnp.float32),
        pltpu.SemaphoreType.DMA,
        pltpu.SemaphoreType.DMA,
    ],
)


def _out_body(s_ref, dis_ref, att_ref, cz_ref, ch_ref,
              linW_ref, linb_ref, o_ref):
    f32 = jnp.float32
    dis = dis_ref[...]
    g = [s_ref[i] * dis for i in range(CH)]
    Zall = jnp.concatenate(g[:CH // 2], axis=1) + cz_ref[...]
    Hall = jnp.concatenate(g[CH // 2:], axis=1) + ch_ref[...]
    AB = (1.0 - jax.nn.sigmoid(Zall)) * jnp.tanh(Hall)
    att = att_ref[...]
    m = jnp.max(att, axis=1, keepdims=True)
    e = jnp.exp(att - m)
    probs = e / jnp.sum(e, axis=1, keepdims=True)
    r12 = lax.broadcasted_iota(jnp.int32, (P * HID, P), 0) // HID
    c12 = lax.broadcasted_iota(jnp.int32, (P * HID, P), 1)
    onehot = (r12 == c12).astype(f32)
    pv = lax.dot_general(onehot, probs, (((1,), (1,)), ((), ())),
                         preferred_element_type=f32)
    ri = lax.broadcasted_iota(jnp.int32, (P * HID, HID), 0) % HID
    ci = lax.broadcasted_iota(jnp.int32, (P * HID, HID), 1)
    Pmat = jnp.where(ri == ci, pv, 0.0)
    H = jnp.dot(AB, Pmat, preferred_element_type=f32)
    o_ref[...] = jnp.dot(jnp.maximum(H, 0.0), linW_ref[...],
                         preferred_element_type=f32) + linb_ref[...]


_out_call = pl.pallas_call(
    _out_body,
    grid=(N // _BN,),
    in_specs=[
        pl.BlockSpec((CH, _BN, CW), lambda i: (0, i, 0)),
        pl.BlockSpec((_BN, 1), lambda i: (i, 0)),
        pl.BlockSpec((1, P), lambda i: (0, 0)),
        pl.BlockSpec((1, P * HID), lambda i: (0, 0)),
        pl.BlockSpec((1, P * HID), lambda i: (0, 0)),
        pl.BlockSpec((HID, P), lambda i: (0, 0)),
        pl.BlockSpec((1, P), lambda i: (0, 0)),
    ],
    out_specs=pl.BlockSpec((_BN, P), lambda i: (i, 0)),
    out_shape=jax.ShapeDtypeStruct((N, P), jnp.float32),
)

def kernel(x, edge_index, att, Wz, bz, Wr, br, Wh, bh,
           LzW, Lzb, LrW, Lrb, LhW, Lhb, linW, linb):
    src = edge_index[0]
    dst = edge_index[1]
    x3 = jnp.transpose(x, (2, 0, 1))
    zrows = jnp.zeros((RPT, CW), jnp.float32)
    zdeg = jnp.zeros((RPT, DW), jnp.float32)
    ones = jnp.ones((EBA, DW), jnp.float32)
    loops = jnp.arange(NPAD, dtype=jnp.int32)
    npadel = EPAD - E - NPAD
    src_e = jnp.concatenate([src, loops, jnp.zeros((npadel,), jnp.int32)])
    dst_e = jnp.concatenate([dst, loops, jnp.full((npadel,), NPAD, jnp.int32)])

    degp = _deg_call(dst, ones, zdeg).reshape(NC, NPAD, DW)
    dis = _dis_call(degp)
    y = _proj_call(x3, dis, Wz, Wh, LzW, LhW)
    yflat = y.reshape(CH * NPAD, CW)
    s_flat = _agg_call(yflat, src_e, dst_e, zrows)
    s4 = s_flat.reshape(CH, NPAD, CW)

    cz = jnp.tile(bz @ LzW[:HID] + Lzb, P).reshape(1, P * HID)
    ch = jnp.tile(bh @ LhW[:HID] + Lhb, P).reshape(1, P * HID)
    return _out_call(s4, dis, att.reshape(1, P), cz, ch,
                     linW, linb.reshape(1, P))


# speedup vs baseline: 1.6383x; 1.6383x over previous
import functools

import jax
import jax.numpy as jnp
from jax import lax
from jax.experimental import pallas as pl
from jax.experimental.pallas import tpu as pltpu
from jax.experimental.pallas import tpu_sc as plsc

N = 10000
E = 320000
F = 128
HID = 32
P = 12

NC = 2
NS = 16

CW = 128
CH = (2 * HID * P) // CW
EB = 80
TE_SC = E // NS
NB_C = TE_SC // EB
TE_A = E // (NC * NS)
NPAD = 10240
RPT = NPAD // NS

_mesh = plsc.VectorSubcoreMesh(
    core_axis_name="c", subcore_axis_name="s", num_cores=NC, num_subcores=NS)


DW = 128
NB_A = TE_A // EB


def _deg_body(dst_hbm, ones_hbm, z_hbm, out_hbm, dstb, dstc, onesb, sdeg):
    c = lax.axis_index("c")
    s = lax.axis_index("s")
    w = s * NC + c
    pltpu.sync_copy(dst_hbm.at[pl.ds(w * TE_A, TE_A)], dstb)
    pltpu.sync_copy(ones_hbm, onesb)
    pltpu.sync_copy(z_hbm, sdeg.at[pl.ds(s * RPT, RPT)])
    plsc.subcore_barrier()

    def astep(b, carry):
        for k in range(EB // 16):
            dstc[pl.ds(k * 16, 16)] = dstb[pl.ds(b * EB + k * 16, 16)]
        pltpu.sync_copy(onesb, sdeg.at[dstc], add=True)
        return carry

    lax.fori_loop(0, NB_A, astep, 0)
    plsc.subcore_barrier()
    pltpu.sync_copy(sdeg.at[pl.ds(s * RPT, RPT)],
                    out_hbm.at[pl.ds(c * NPAD + s * RPT, RPT)])


_deg_call = pl.kernel(
    _deg_body,
    out_type=jax.ShapeDtypeStruct((NC * NPAD, DW), jnp.float32),
    mesh=_mesh,
    scratch_types=[
        pltpu.VMEM((TE_A,), jnp.int32),
        pltpu.VMEM((EB,), jnp.int32),
        pltpu.VMEM((EB, DW), jnp.float32),
        pltpu.VMEM_SHARED((NPAD, DW), jnp.float32),
    ],
)


def _dis_body(deg_ref, dis_ref):
    deg = deg_ref[0, 0:N, 0:1] + deg_ref[1, 0:N, 0:1]
    dis_ref[...] = lax.rsqrt(deg + 1.0)


_dis_call = pl.pallas_call(
    _dis_body,
    out_shape=jax.ShapeDtypeStruct((N, 1), jnp.float32),
)


_BN = 1000


def _proj_body(x_ref, dis_ref, Wz_ref, Wh_ref, LzW_ref, LhW_ref, y_ref):
    f32 = jnp.float32
    Wz2 = jnp.dot(Wz_ref[...], LzW_ref[0:HID, :], preferred_element_type=f32)
    Wh2 = jnp.dot(Wh_ref[...], LhW_ref[0:HID, :], preferred_element_type=f32)
    dis = dis_ref[...]
    zs = [jnp.dot(x_ref[p], Wz2, preferred_element_type=f32) for p in range(P)]
    hs = [jnp.dot(x_ref[p], Wh2, preferred_element_type=f32) for p in range(P)]
    for i in range(CH // 2):
        y_ref[i] = dis * jnp.concatenate(zs[4 * i:4 * i + 4], axis=1)
        y_ref[CH // 2 + i] = dis * jnp.concatenate(hs[4 * i:4 * i + 4], axis=1)


_proj_call = pl.pallas_call(
    _proj_body,
    grid=(N // _BN,),
    in_specs=[
        pl.BlockSpec((P, _BN, F), lambda i: (0, i, 0)),
        pl.BlockSpec((_BN, 1), lambda i: (i, 0)),
        pl.BlockSpec((F, HID), lambda i: (0, 0)),
        pl.BlockSpec((F, HID), lambda i: (0, 0)),
        pl.BlockSpec((2 * HID, HID), lambda i: (0, 0)),
        pl.BlockSpec((2 * HID, HID), lambda i: (0, 0)),
    ],
    out_specs=pl.BlockSpec((CH, _BN, CW), lambda i: (0, i, 0)),
    out_shape=jax.ShapeDtypeStruct((CH, NPAD, CW), jnp.float32),
)


SB = 4000
NSB = TE_SC // SB
NB_I = SB // EB
NTRI = (NB_I - 2) // 3


def _agg_body(y_hbm, src_hbm, dst_hbm, z_hbm, out_hbm,
              srcb, dstb, rows0, rows1, rows2, idx0, idx1, idx2,
              dstc0, dstc1, dstc2, ssh, sem0, sem1, sem2):
    c = lax.axis_index("c")
    s = lax.axis_index("s")
    for q in range(CH // NC):
        chunk = c * (CH // NC) + q
        off = chunk * NPAD
        pltpu.sync_copy(z_hbm, ssh.at[pl.ds(s * RPT, RPT)])
        plsc.subcore_barrier()

        def fill(b, idxr, dstcr):
            for k in range(EB // 16):
                idxr[pl.ds(k * 16, 16)] = srcb[pl.ds(b * EB + k * 16, 16)] + off
                dstcr[pl.ds(k * 16, 16)] = dstb[pl.ds(b * EB + k * 16, 16)]

        def sbstep(sb, carry):
            base = s * TE_SC + sb * SB
            pltpu.sync_copy(src_hbm.at[pl.ds(base, SB)], srcb)
            pltpu.sync_copy(dst_hbm.at[pl.ds(base, SB)], dstb)
            fill(0, idx0, dstc0)
            pltpu.async_copy(y_hbm.at[idx0], rows0, sem0)
            fill(1, idx1, dstc1)
            pltpu.async_copy(y_hbm.at[idx1], rows1, sem1)

            def triple(bb, carry2):
                b0 = bb * 3
                fill(b0 + 2, idx2, dstc2)
                pltpu.async_copy(y_hbm.at[idx2], rows2, sem2)
                pltpu.make_async_copy(y_hbm.at[idx0], rows0, sem0).wait()
                pltpu.sync_copy(rows0, ssh.at[dstc0], add=True)
                fill(b0 + 3, idx0, dstc0)
                pltpu.async_copy(y_hbm.at[idx0], rows0, sem0)
                pltpu.make_async_copy(y_hbm.at[idx1], rows1, sem1).wait()
                pltpu.sync_copy(rows1, ssh.at[dstc1], add=True)
                fill(b0 + 4, idx1, dstc1)
                pltpu.async_copy(y_hbm.at[idx1], rows1, sem1)
                pltpu.make_async_copy(y_hbm.at[idx2], rows2, sem2).wait()
                pltpu.sync_copy(rows2, ssh.at[dstc2], add=True)
                return carry2

            lax.fori_loop(0, NTRI, triple, 0)
            pltpu.make_async_copy(y_hbm.at[idx0], rows0, sem0).wait()
            pltpu.sync_copy(rows0, ssh.at[dstc0], add=True)
            pltpu.make_async_copy(y_hbm.at[idx1], rows1, sem1).wait()
            pltpu.sync_copy(rows1, ssh.at[dstc1], add=True)
            return carry

        lax.fori_loop(0, NSB, sbstep, 0)
        plsc.subcore_barrier()
        pltpu.sync_copy(ssh.at[pl.ds(s * RPT, RPT)],
                        out_hbm.at[pl.ds(off + s * RPT, RPT)])
        plsc.subcore_barrier()


_agg_call = pl.kernel(
    _agg_body,
    out_type=jax.ShapeDtypeStruct((CH * NPAD, CW), jnp.float32),
    mesh=_mesh,
    scratch_types=[
        pltpu.VMEM((SB,), jnp.int32),
        pltpu.VMEM((SB,), jnp.int32),
        pltpu.VMEM((EB, CW), jnp.float32),
        pltpu.VMEM((EB, CW), jnp.float32),
        pltpu.VMEM((EB, CW), jnp.float32),
        pltpu.VMEM((EB,), jnp.int32),
        pltpu.VMEM((EB,), jnp.int32),
        pltpu.VMEM((EB,), jnp.int32),
        pltpu.VMEM((EB,), jnp.int32),
        pltpu.VMEM((EB,), jnp.int32),
        pltpu.VMEM((EB,), jnp.int32),
        pltpu.VMEM_SHARED((NPAD, CW), jnp.float32),
        pltpu.SemaphoreType.DMA,
        pltpu.SemaphoreType.DMA,
        pltpu.SemaphoreType.DMA,
    ],
)


def _out_body(s_ref, y_ref, dis_ref, att_ref, cz_ref, ch_ref,
              linW_ref, linb_ref, o_ref):
    f32 = jnp.float32
    dis = dis_ref[...]
    g = [(s_ref[i] + y_ref[i]) * dis for i in range(CH)]
    Zall = jnp.concatenate(g[:CH // 2], axis=1) + cz_ref[...]
    Hall = jnp.concatenate(g[CH // 2:], axis=1) + ch_ref[...]
    AB = (1.0 - jax.nn.sigmoid(Zall)) * jnp.tanh(Hall)
    att = att_ref[...]
    m = jnp.max(att, axis=1, keepdims=True)
    e = jnp.exp(att - m)
    probs = e / jnp.sum(e, axis=1, keepdims=True)
    r12 = lax.broadcasted_iota(jnp.int32, (P * HID, P), 0) // HID
    c12 = lax.broadcasted_iota(jnp.int32, (P * HID, P), 1)
    onehot = (r12 == c12).astype(f32)
    pv = lax.dot_general(onehot, probs, (((1,), (1,)), ((), ())),
                         preferred_element_type=f32)
    ri = lax.broadcasted_iota(jnp.int32, (P * HID, HID), 0) % HID
    ci = lax.broadcasted_iota(jnp.int32, (P * HID, HID), 1)
    Pmat = jnp.where(ri == ci, pv, 0.0)
    H = jnp.dot(AB, Pmat, preferred_element_type=f32)
    o_ref[...] = jnp.dot(jnp.maximum(H, 0.0), linW_ref[...],
                         preferred_element_type=f32) + linb_ref[...]


_out_call = pl.pallas_call(
    _out_body,
    grid=(N // _BN,),
    in_specs=[
        pl.BlockSpec((CH, _BN, CW), lambda i: (0, i, 0)),
        pl.BlockSpec((CH, _BN, CW), lambda i: (0, i, 0)),
        pl.BlockSpec((_BN, 1), lambda i: (i, 0)),
        pl.BlockSpec((1, P), lambda i: (0, 0)),
        pl.BlockSpec((1, P * HID), lambda i: (0, 0)),
        pl.BlockSpec((1, P * HID), lambda i: (0, 0)),
        pl.BlockSpec((HID, P), lambda i: (0, 0)),
        pl.BlockSpec((1, P), lambda i: (0, 0)),
    ],
    out_specs=pl.BlockSpec((_BN, P), lambda i: (i, 0)),
    out_shape=jax.ShapeDtypeStruct((N, P), jnp.float32),
)


def kernel(x, edge_index, att, Wz, bz, Wr, br, Wh, bh,
           LzW, Lzb, LrW, Lrb, LhW, Lhb, linW, linb):
    src = edge_index[0]
    dst = edge_index[1]
    x3 = jnp.transpose(x, (2, 0, 1))
    zrows = jnp.zeros((RPT, CW), jnp.float32)
    zdeg = jnp.zeros((RPT, DW), jnp.float32)
    ones = jnp.ones((EB, DW), jnp.float32)

    degp = _deg_call(dst, ones, zdeg).reshape(NC, NPAD, DW)
    dis = _dis_call(degp)
    y = _proj_call(x3, dis, Wz, Wh, LzW, LhW)
    yflat = y.reshape(CH * NPAD, CW)
    s_flat = _agg_call(yflat, src, dst, zrows)
    s4 = s_flat.reshape(CH, NPAD, CW)

    cz = jnp.tile(bz @ LzW[:HID] + Lzb, P).reshape(1, P * HID)
    ch = jnp.tile(bh @ LhW[:HID] + Lhb, P).reshape(1, P * HID)
    return _out_call(s4, y, dis, att.reshape(1, P), cz, ch,
                     linW, linb.reshape(1, P))
